# baseline (speedup 1.0000x reference)
import functools

import jax
import jax.numpy as jnp
import numpy as np
from jax import lax
from jax.experimental import pallas as pl
from jax.experimental.pallas import tpu as pltpu
from jax.experimental.pallas import tpu_sc as plsc

N_TOKENS = 1024
K_CODES = 1024
D_MODEL = 1024
MASK_TOKEN_ID = K_CODES
CHOICE_TEMPERATURE = 4.5
RATIO = 0.5

_MASK_RATIO = 0.5 * (1.0 + np.cos(np.pi * RATIO))
_TEMPERATURE = float(CHOICE_TEMPERATURE * (1.0 - _MASK_RATIO))
_MASK_RATIO_Z = 0.0 if _MASK_RATIO < 1e-08 else _MASK_RATIO
_MASK_LEN = int(np.floor(512 * _MASK_RATIO_Z))

_NC = 2
_NS = 16
_NW = _NC * _NS
_BPW = N_TOKENS // _NW


def _sc_gather_body(emb_hbm, z_hbm, m_hbm, out_hbm, z_v, m_v, idx_v, rows_v, sem):
    wid = lax.axis_index("s") * _NC + lax.axis_index("c")
    base = wid * _BPW
    pltpu.sync_copy(z_hbm.at[pl.ds(base, _BPW)], z_v)
    pltpu.sync_copy(m_hbm.at[pl.ds(base, _BPW)], m_v)
    for k in range(_BPW // 16):
        zk = z_v[pl.ds(k * 16, 16)]
        mk = m_v[pl.ds(k * 16, 16)]
        idx_v[pl.ds(k * 16, 16)] = jnp.where(mk != 0, MASK_TOKEN_ID, zk)
    pltpu.async_copy(emb_hbm.at[idx_v], rows_v, sem).wait()
    pltpu.sync_copy(rows_v, out_hbm.at[pl.ds(base, _BPW)])


@functools.cache
def _sc_gather():
    return pl.kernel(
        _sc_gather_body,
        out_type=jax.ShapeDtypeStruct((N_TOKENS, D_MODEL), jnp.float32),
        mesh=plsc.VectorSubcoreMesh(core_axis_name="c", subcore_axis_name="s"),
        scratch_types=[
            pltpu.VMEM((_BPW,), jnp.int32),
            pltpu.VMEM((_BPW,), jnp.int32),
            pltpu.VMEM((_BPW,), jnp.int32),
            pltpu.VMEM((_BPW, D_MODEL), jnp.float32),
            pltpu.SemaphoreType.DMA,
        ],
    )


def _tc_body(h_ref, w_ref, b_ref, z_ref, mrow_ref, grow_ref, mcol_ref, gcol_ref,
             zp_ref, msel_ref, conf_ref):
    n = N_TOKENS
    logits = jnp.dot(h_ref[...], w_ref[...], preferred_element_type=jnp.float32)
    logits = logits + b_ref[...]
    m = jnp.max(logits, axis=1, keepdims=True)
    e = jnp.exp(logits - m)
    s = jnp.sum(e, axis=1, keepdims=True)
    q = e / s
    zpp_col = jnp.max(q, axis=1, keepdims=True)
    ii = lax.broadcasted_iota(jnp.int32, (n, n), 1)
    jj = lax.broadcasted_iota(jnp.int32, (n, n), 0)
    am_col = jnp.min(jnp.where(q == zpp_col, ii, n), axis=1, keepdims=True)

    eye = ii == jj
    ones_row = jnp.ones((1, n), jnp.float32)
    zpp_row = jnp.dot(ones_row, jnp.where(eye, zpp_col, 0.0),
                      precision=lax.Precision.HIGHEST,
                      preferred_element_type=jnp.float32)
    am_row = jnp.dot(ones_row, jnp.where(eye, am_col.astype(jnp.float32), 0.0),
                     precision=lax.Precision.HIGHEST,
                     preferred_element_type=jnp.float32).astype(jnp.int32)

    mrow = mrow_ref[...] != 0
    zp_ref[...] = jnp.where(mrow, am_row, z_ref[...])
    conf_row = jnp.where(mrow, zpp_row + _TEMPERATURE * grow_ref[...], jnp.inf)
    conf_ref[...] = conf_row
    conf_col = jnp.where(mcol_ref[...] != 0, zpp_col + _TEMPERATURE * gcol_ref[...],
                         jnp.inf)

    cmp = (conf_col < conf_row) | ((conf_col == conf_row) & (jj < ii))
    rank_row = jnp.sum(cmp.astype(jnp.int32), axis=0, keepdims=True)
    msel_ref[...] = ((rank_row < _MASK_LEN) & mrow).astype(jnp.int32)


def kernel(z_indices, mask_b, mask_num, emb, W, b):
    del mask_num
    g_row = jax.random.gumbel(jax.random.key(42), (1, N_TOKENS), jnp.float32)
    mask_i = mask_b.astype(jnp.int32)
    h = _sc_gather()(emb, z_indices.reshape(N_TOKENS), mask_i.reshape(N_TOKENS))
    zp, msel, conf = pl.pallas_call(
        _tc_body,
        out_shape=(
            jax.ShapeDtypeStruct((1, N_TOKENS), jnp.int32),
            jax.ShapeDtypeStruct((1, N_TOKENS), jnp.int32),
            jax.ShapeDtypeStruct((1, N_TOKENS), jnp.float32),
        ),
    )(h, W, b.reshape(1, K_CODES), z_indices, mask_i, g_row,
      mask_i.reshape(N_TOKENS, 1), g_row.reshape(N_TOKENS, 1))
    return zp, msel.astype(bool), conf


# speedup vs baseline: 4.2027x; 4.2027x over previous
import jax
import jax.numpy as jnp
import numpy as np
from jax import lax
from jax.experimental import pallas as pl

N_TOKENS = 1024
K_CODES = 1024
D_MODEL = 1024
MASK_TOKEN_ID = K_CODES
CHOICE_TEMPERATURE = 4.5
RATIO = 0.5

_MASK_RATIO = 0.5 * (1.0 + np.cos(np.pi * RATIO))
_TEMPERATURE = float(CHOICE_TEMPERATURE * (1.0 - _MASK_RATIO))
_MASK_RATIO_Z = 0.0 if _MASK_RATIO < 1e-08 else _MASK_RATIO
_MASK_LEN = int(np.floor(512 * _MASK_RATIO_Z))


def _tc_body(er_ref, w_ref, b_ref, z_ref, mrow_ref, mcol_ref, grow_ref, gcol_ref,
             zp_ref, msel_ref, conf_ref):
    n = N_TOKENS
    p = jnp.dot(er_ref[...], w_ref[...], preferred_element_type=jnp.float32)
    p = p + b_ref[...]
    m = jnp.max(p, axis=1, keepdims=True)
    e = jnp.exp(p - m)
    s = jnp.sum(e, axis=1, keepdims=True)
    q = e / s
    zpp = jnp.max(q, axis=1, keepdims=True)
    kk = lax.broadcasted_iota(jnp.int32, (1, K_CODES), 1)
    am = jnp.min(jnp.where(q == zpp, kk, K_CODES), axis=1, keepdims=True)

    mrow = mrow_ref[...] != 0
    zp_ref[...] = jnp.where(mrow, am, z_ref[...])
    conf_row = jnp.where(mrow, zpp + _TEMPERATURE * grow_ref[...], jnp.inf)
    conf_col = jnp.where(mcol_ref[...] != 0, zpp + _TEMPERATURE * gcol_ref[...],
                         jnp.inf)
    conf_ref[...] = conf_row
    ii = lax.broadcasted_iota(jnp.int32, (n, n), 1)
    jj = lax.broadcasted_iota(jnp.int32, (n, n), 0)
    cmp = (conf_col < conf_row) | ((conf_col == conf_row) & (jj < ii))
    rank_row = jnp.sum(cmp.astype(jnp.int32), axis=0, keepdims=True)
    msel_ref[...] = ((rank_row < _MASK_LEN) & mrow).astype(jnp.int32)


def kernel(z_indices, mask_b, mask_num, emb, W, b):
    del mask_num
    g_row = jax.random.gumbel(jax.random.key(42), (1, N_TOKENS), jnp.float32)
    mask_i = mask_b.astype(jnp.int32)
    zp, msel, conf = pl.pallas_call(
        _tc_body,
        out_shape=(
            jax.ShapeDtypeStruct((1, N_TOKENS), jnp.int32),
            jax.ShapeDtypeStruct((1, N_TOKENS), jnp.int32),
            jax.ShapeDtypeStruct((1, N_TOKENS), jnp.float32),
        ),
    )(emb[MASK_TOKEN_ID:MASK_TOKEN_ID + 1], W, b.reshape(1, K_CODES),
      z_indices, mask_i, mask_i.reshape(N_TOKENS, 1), g_row,
      g_row.reshape(N_TOKENS, 1))
    return zp, msel.astype(bool), conf


# speedup vs baseline: 5.7218x; 1.3614x over previous
import functools

import jax
import jax.numpy as jnp
import numpy as np
from jax import lax
from jax.experimental import pallas as pl
from jax.experimental.pallas import tpu as pltpu

N_TOKENS = 1024
K_CODES = 1024
D_MODEL = 1024
MASK_TOKEN_ID = K_CODES
CHOICE_TEMPERATURE = 4.5
RATIO = 0.5

_MASK_RATIO = 0.5 * (1.0 + np.cos(np.pi * RATIO))
_TEMPERATURE = float(CHOICE_TEMPERATURE * (1.0 - _MASK_RATIO))
_MASK_RATIO_Z = 0.0 if _MASK_RATIO < 1e-08 else _MASK_RATIO
_MASK_LEN = int(np.floor(512 * _MASK_RATIO_Z))

_KCH = 4
_KW = K_CODES // _KCH


@functools.cache
def _gumbel_const():
    with jax.ensure_compile_time_eval():
        g = jax.random.gumbel(jax.random.key(42), (1, N_TOKENS), jnp.float32)
    return np.asarray(g)


def _tc_body(er_ref, w_ref, b_ref, z_ref, mrow_ref, mcol_ref, grow_ref, gcol_ref,
             zp_ref, msel_ref, conf_ref, p_scr):
    n = N_TOKENS
    k = pl.program_id(0)
    chunk = jnp.dot(er_ref[0:1, :], w_ref[...], preferred_element_type=jnp.float32)
    p_scr[:, pl.ds(k * _KW, _KW)] = chunk + b_ref[...]

    @pl.when(k == _KCH - 1)
    def _():
        p = p_scr[...]
        m = jnp.max(p, axis=1, keepdims=True)
        e = jnp.exp(p - m)
        s = jnp.sum(e, axis=1, keepdims=True)
        q = e / s
        zpp = jnp.max(q, axis=1, keepdims=True)
        kk = lax.broadcasted_iota(jnp.int32, (1, K_CODES), 1)
        am = jnp.min(jnp.where(q == zpp, kk, K_CODES), axis=1, keepdims=True)

        mrow = mrow_ref[...] != 0
        zp_ref[...] = jnp.where(mrow, am, z_ref[...])
        conf_row = jnp.where(mrow, zpp + _TEMPERATURE * grow_ref[...], jnp.inf)
        conf_col = jnp.where(mcol_ref[...] != 0, zpp + _TEMPERATURE * gcol_ref[...],
                             jnp.inf)
        conf_ref[...] = conf_row
        ii = lax.broadcasted_iota(jnp.int32, (n, n), 1)
        jj = lax.broadcasted_iota(jnp.int32, (n, n), 0)
        cmp = (conf_col < conf_row) | ((conf_col == conf_row) & (jj < ii))
        rank_row = jnp.sum(cmp.astype(jnp.int32), axis=0, keepdims=True)
        msel_ref[...] = ((rank_row < _MASK_LEN) & mrow).astype(jnp.int32)


def kernel(z_indices, mask_b, mask_num, emb, W, b):
    del mask_num
    g_row = jnp.asarray(_gumbel_const())
    mask_i = mask_b.astype(jnp.int32)
    zp, msel, conf = pl.pallas_call(
        _tc_body,
        grid=(_KCH,),
        in_specs=[
            pl.BlockSpec((8, D_MODEL), lambda k: (MASK_TOKEN_ID // 8, 0)),
            pl.BlockSpec((D_MODEL, _KW), lambda k: (0, k)),
            pl.BlockSpec((1, _KW), lambda k: (0, k)),
            pl.BlockSpec((1, N_TOKENS), lambda k: (0, 0)),
            pl.BlockSpec((1, N_TOKENS), lambda k: (0, 0)),
            pl.BlockSpec((N_TOKENS, 1), lambda k: (0, 0)),
            pl.BlockSpec((1, N_TOKENS), lambda k: (0, 0)),
            pl.BlockSpec((N_TOKENS, 1), lambda k: (0, 0)),
        ],
        out_specs=[
            pl.BlockSpec((1, N_TOKENS), lambda k: (0, 0)),
            pl.BlockSpec((1, N_TOKENS), lambda k: (0, 0)),
            pl.BlockSpec((1, N_TOKENS), lambda k: (0, 0)),
        ],
        out_shape=(
            jax.ShapeDtypeStruct((1, N_TOKENS), jnp.int32),
            jax.ShapeDtypeStruct((1, N_TOKENS), jnp.int32),
            jax.ShapeDtypeStruct((1, N_TOKENS), jnp.float32),
        ),
        scratch_shapes=[pltpu.VMEM((1, K_CODES), jnp.float32)],
    )(emb, W, b.reshape(1, K_CODES), z_indices, mask_i,
      mask_i.reshape(N_TOKENS, 1), g_row, g_row.reshape(N_TOKENS, 1))
    return zp, msel.astype(bool), conf
